# Initial kernel scaffold; baseline (speedup 1.0000x reference)
import jax
import jax.numpy as jnp
from jax.experimental import pallas as pl


def kernel(x, ln_pre_w, router_w, shared_w12, shared_w3, expert_w12, expert_w3, ln_post_w):
    raise NotImplementedError("write your pallas kernel here")


# speedup vs baseline: 1.0845x; 1.0845x over previous
import functools

import jax
import jax.numpy as jnp
from jax.experimental import pallas as pl
from jax.experimental.pallas import tpu as pltpu

ENC_DIM = 1280
K = 2
IN_DIM = ENC_DIM * K
OUT_DIM = 4096
NUM_EXPERTS = 8
TOP_K = 4
ROUTED_HIDDEN = 2048
SHARED_HIDDEN = 2048
EPS = 1e-6
NT = 4096
LANES = 128


def _norm_body(x_ref, w_ref, rw_ref, nx_ref, lg_ref):
    x = x_ref[...]
    var = jnp.mean(x * x, axis=-1, keepdims=True)
    nx = x * jax.lax.rsqrt(var + EPS) * w_ref[...]
    nx_ref[...] = nx.astype(jnp.bfloat16)
    lg_ref[...] = jax.lax.dot_general(
        nx, rw_ref[...], (((1,), (1,)), ((), ())),
        preferred_element_type=jnp.float32)


def _k_norm(tokens, ln_pre_w, router_w_pad):
    rt = 16
    bt = NT // rt
    return pl.pallas_call(
        _norm_body,
        grid=(rt,),
        in_specs=[
            pl.BlockSpec((bt, IN_DIM), lambda i: (i, 0)),
            pl.BlockSpec((1, IN_DIM), lambda i: (0, 0)),
            pl.BlockSpec((LANES, IN_DIM), lambda i: (0, 0)),
        ],
        out_specs=[
            pl.BlockSpec((bt, IN_DIM), lambda i: (i, 0)),
            pl.BlockSpec((bt, LANES), lambda i: (i, 0)),
        ],
        out_shape=[
            jax.ShapeDtypeStruct((NT, IN_DIM), jnp.bfloat16),
            jax.ShapeDtypeStruct((NT, LANES), jnp.float32),
        ],
    )(tokens, ln_pre_w.reshape(1, IN_DIM), router_w_pad)


def _route_body(lg_ref, wdense_ref, aux_ref):
    lg = lg_ref[...]
    lane = jax.lax.broadcasted_iota(jnp.int32, lg.shape, 1)
    valid = lane < NUM_EXPERTS
    neg = jnp.float32(-1e30)
    lg = jnp.where(valid, lg, neg)
    m = jnp.max(lg, axis=-1, keepdims=True)
    e = jnp.where(valid, jnp.exp(lg - m), 0.0)
    p = e / jnp.sum(e, axis=-1, keepdims=True)

    imp = jnp.sum(p, axis=0)
    aux = jnp.sum(imp * imp) / (NT * NT) * NUM_EXPERTS
    aux_ref[0, 0] = aux

    work = p
    wdense = jnp.zeros_like(p)
    wsum = jnp.zeros((p.shape[0], 1), jnp.float32)
    sel = []
    for _ in range(TOP_K):
        cur = jnp.max(work, axis=-1, keepdims=True)
        idx = jnp.min(jnp.where(work == cur, lane, LANES), axis=-1,
                      keepdims=True)
        sel.append((idx, cur))
        wsum = wsum + cur
        work = jnp.where(lane == idx, neg, work)
    inv = 1.0 / (wsum + 1e-20)
    for idx, cur in sel:
        wdense = wdense + jnp.where(lane == idx, cur * inv, 0.0)
    wdense_ref[...] = wdense


def _k_route(logits):
    return pl.pallas_call(
        _route_body,
        out_shape=[
            jax.ShapeDtypeStruct((NT, LANES), jnp.float32),
            jax.ShapeDtypeStruct((1, 1), jnp.float32),
        ],
        out_specs=[
            pl.BlockSpec((NT, LANES), lambda: (0, 0)),
            pl.BlockSpec(memory_space=pltpu.SMEM),
        ],
    )(logits)


def _s1_body(nx_ref, wg_ref, wv_ref, h_ref):
    nx = nx_ref[...]
    g = jax.lax.dot_general(nx, wg_ref[...], (((1,), (1,)), ((), ())),
                            preferred_element_type=jnp.float32)
    v = jax.lax.dot_general(nx, wv_ref[...], (((1,), (1,)), ((), ())),
                            preferred_element_type=jnp.float32)
    h_ref[...] = (g * jax.lax.logistic(g) * v).astype(jnp.bfloat16)


def _k_s1(nx, w12):
    rt, bh = 8, 1024
    bt = NT // rt
    return pl.pallas_call(
        _s1_body,
        grid=(SHARED_HIDDEN // bh, rt),
        in_specs=[
            pl.BlockSpec((bt, IN_DIM), lambda c, i: (i, 0)),
            pl.BlockSpec((bh, IN_DIM), lambda c, i: (c, 0)),
            pl.BlockSpec((bh, IN_DIM),
                         lambda c, i: (SHARED_HIDDEN // bh + c, 0)),
        ],
        out_specs=pl.BlockSpec((bt, bh), lambda c, i: (i, c)),
        out_shape=jax.ShapeDtypeStruct((NT, SHARED_HIDDEN), jnp.bfloat16),
    )(nx, w12, w12)


def _s2_body(h_ref, w3_ref, o_ref):
    o_ref[...] = jax.lax.dot_general(
        h_ref[...], w3_ref[...], (((1,), (1,)), ((), ())),
        preferred_element_type=jnp.float32)


def _k_s2(h, w3):
    rt, bo = 8, 1024
    bt = NT // rt
    return pl.pallas_call(
        _s2_body,
        grid=(rt, OUT_DIM // bo),
        in_specs=[
            pl.BlockSpec((bt, SHARED_HIDDEN), lambda i, c: (i, 0)),
            pl.BlockSpec((bo, SHARED_HIDDEN), lambda i, c: (c, 0)),
        ],
        out_specs=pl.BlockSpec((bt, bo), lambda i, c: (i, c)),
        out_shape=jax.ShapeDtypeStruct((NT, OUT_DIM), jnp.float32),
    )(h, w3)


def _r1_body(nx_ref, wd_ref, wg_ref, wv_ref, h_ref):
    e = pl.program_id(0)
    nx = nx_ref[...]
    g = jax.lax.dot_general(nx, wg_ref[0], (((1,), (1,)), ((), ())),
                            preferred_element_type=jnp.float32)
    v = jax.lax.dot_general(nx, wv_ref[0], (((1,), (1,)), ((), ())),
                            preferred_element_type=jnp.float32)
    wd = wd_ref[...]
    lane = jax.lax.broadcasted_iota(jnp.int32, wd.shape, 1)
    w = jnp.sum(jnp.where(lane == e, wd, 0.0), axis=1, keepdims=True)
    h_ref[0] = ((g * jax.lax.logistic(g) * v) * w).astype(jnp.bfloat16)


def _k_r1(nx, wdense, ew12):
    rt, bh = 8, 1024
    bt = NT // rt
    nh = ROUTED_HIDDEN // bh
    return pl.pallas_call(
        _r1_body,
        grid=(NUM_EXPERTS, nh, rt),
        in_specs=[
            pl.BlockSpec((bt, IN_DIM), lambda e, c, i: (i, 0)),
            pl.BlockSpec((bt, LANES), lambda e, c, i: (i, 0)),
            pl.BlockSpec((1, bh, IN_DIM), lambda e, c, i: (e, c, 0)),
            pl.BlockSpec((1, bh, IN_DIM), lambda e, c, i: (e, nh + c, 0)),
        ],
        out_specs=pl.BlockSpec((1, bt, bh), lambda e, c, i: (e, i, c)),
        out_shape=jax.ShapeDtypeStruct((NUM_EXPERTS, NT, ROUTED_HIDDEN),
                                       jnp.bfloat16),
    )(nx, wdense, ew12, ew12)


def _r2_body(h_ref, w3_ref, o_ref):
    e = pl.program_id(2)

    @pl.when(e == 0)
    def _():
        o_ref[...] = jnp.zeros_like(o_ref)

    o_ref[...] += jax.lax.dot_general(
        h_ref[0], w3_ref[0], (((1,), (1,)), ((), ())),
        preferred_element_type=jnp.float32)


def _k_r2(h, ew3):
    rt, bo = 8, 1024
    bt = NT // rt
    return pl.pallas_call(
        _r2_body,
        grid=(rt, OUT_DIM // bo, NUM_EXPERTS),
        in_specs=[
            pl.BlockSpec((1, bt, ROUTED_HIDDEN), lambda i, c, e: (e, i, 0)),
            pl.BlockSpec((1, bo, ROUTED_HIDDEN), lambda i, c, e: (e, c, 0)),
        ],
        out_specs=pl.BlockSpec((bt, bo), lambda i, c, e: (i, c)),
        out_shape=jax.ShapeDtypeStruct((NT, OUT_DIM), jnp.float32),
    )(h, ew3)


def _final_body(s_ref, r_ref, w_ref, o_ref):
    y = s_ref[...] + r_ref[...]
    var = jnp.mean(y * y, axis=-1, keepdims=True)
    y = y * jax.lax.rsqrt(var + EPS) * w_ref[...]
    o_ref[...] = jnp.clip(y, -30.0, 30.0)


def _k_final(shared, routed, ln_post_w):
    rt = 16
    bt = NT // rt
    return pl.pallas_call(
        _final_body,
        grid=(rt,),
        in_specs=[
            pl.BlockSpec((bt, OUT_DIM), lambda i: (i, 0)),
            pl.BlockSpec((bt, OUT_DIM), lambda i: (i, 0)),
            pl.BlockSpec((1, OUT_DIM), lambda i: (0, 0)),
        ],
        out_specs=pl.BlockSpec((bt, OUT_DIM), lambda i: (i, 0)),
        out_shape=jax.ShapeDtypeStruct((NT, OUT_DIM), jnp.float32),
    )(shared, routed, ln_post_w.reshape(1, OUT_DIM))


def kernel(x, ln_pre_w, router_w, shared_w12, shared_w3, expert_w12,
           expert_w3, ln_post_w):
    B, S, D = x.shape
    tokens = x.reshape(B * S // K, D * K)

    router_w_pad = jnp.zeros((LANES, IN_DIM), jnp.float32).at[:NUM_EXPERTS].set(
        router_w)
    nx, logits = _k_norm(tokens, ln_pre_w, router_w_pad)
    wdense, aux = _k_route(logits)

    hs = _k_s1(nx, shared_w12.astype(jnp.bfloat16))
    shared_out = _k_s2(hs, shared_w3.astype(jnp.bfloat16))

    hr = _k_r1(nx, wdense, expert_w12.astype(jnp.bfloat16))
    routed_out = _k_r2(hr, expert_w3.astype(jnp.bfloat16))

    final = _k_final(shared_out, routed_out, ln_post_w)
    return final.reshape(B, S // K, OUT_DIM), aux[0, 0]
